# speedup vs baseline: 4.8542x; 4.8542x over previous
import functools

import jax
import jax.numpy as jnp
from jax import lax
from jax.experimental import pallas as pl
from jax.experimental.pallas import tpu as pltpu
from jax.experimental.pallas import tpu_sc as plsc

BATCH = 1024
HIST = 200
D = 128
B = BATCH * HIST
NC = 2
NS = 16
NW = NC * NS
B_PER_W = B // NW
CHUNK = 128
N_CHUNKS = B_PER_W // CHUNK

_mesh = plsc.VectorSubcoreMesh(core_axis_name="c", subcore_axis_name="s")


@functools.partial(
    pl.kernel,
    mesh=_mesh,
    out_type=jax.ShapeDtypeStruct((B, D), jnp.float32),
    scratch_types=[
        pltpu.VMEM((CHUNK,), jnp.int32),
        pltpu.VMEM((CHUNK, D), jnp.float32),
        pltpu.SemaphoreType.DMA,
    ],
)
def _embed(idx_hbm, table_hbm, out_hbm, idx_v, rows_v, sem):
    wid = lax.axis_index("s") * NC + lax.axis_index("c")
    base = wid * B_PER_W

    def body(i, carry):
        off = base + i * CHUNK
        pltpu.sync_copy(idx_hbm.at[pl.ds(off, CHUNK)], idx_v)
        pltpu.async_copy(table_hbm.at[idx_v], rows_v, sem).wait()
        pltpu.sync_copy(rows_v, out_hbm.at[pl.ds(off, CHUNK)])
        return carry

    lax.fori_loop(0, N_CHUNKS, body, 0)


def kernel(word_ids, table):
    idx = word_ids.reshape(B).astype(jnp.int32)
    out = _embed(idx, table)
    return out.reshape(BATCH, HIST, D)


# speedup vs baseline: 7.8145x; 1.6098x over previous
import functools

import jax
import jax.numpy as jnp
from jax import lax
from jax.experimental import pallas as pl
from jax.experimental.pallas import tpu as pltpu
from jax.experimental.pallas import tpu_sc as plsc

BATCH = 1024
HIST = 200
D = 128
B = BATCH * HIST
NC = 2
NS = 16
NW = NC * NS
B_PER_W = B // NW
CHUNK = 128
N_CHUNKS = B_PER_W // CHUNK
NBUF = 4

_mesh = plsc.VectorSubcoreMesh(core_axis_name="c", subcore_axis_name="s")


@functools.partial(
    pl.kernel,
    mesh=_mesh,
    out_type=jax.ShapeDtypeStruct((B, D), jnp.float32),
    scratch_types=(
        [pltpu.VMEM((CHUNK,), jnp.int32) for _ in range(NBUF)]
        + [pltpu.VMEM((CHUNK, D), jnp.float32) for _ in range(NBUF)]
        + [pltpu.SemaphoreType.DMA for _ in range(2 * NBUF)]
    ),
)
def _embed(idx_hbm, table_hbm, out_hbm, *scr):
    idx_bufs = scr[0:NBUF]
    row_bufs = scr[NBUF:2 * NBUF]
    gsems = scr[2 * NBUF:3 * NBUF]
    ssems = scr[3 * NBUF:4 * NBUF]

    wid = lax.axis_index("s") * NC + lax.axis_index("c")
    base = wid * B_PER_W

    def fire(c, b):
        off = base + c * CHUNK
        pltpu.sync_copy(idx_hbm.at[pl.ds(off, CHUNK)], idx_bufs[b])
        pltpu.async_copy(table_hbm.at[idx_bufs[b]], row_bufs[b], gsems[b])

    def wait_gather(b):
        pltpu.make_async_copy(
            table_hbm.at[idx_bufs[b]], row_bufs[b], gsems[b]).wait()

    def start_store(c, b):
        off = base + c * CHUNK
        pltpu.async_copy(row_bufs[b], out_hbm.at[pl.ds(off, CHUNK)], ssems[b])

    def wait_store(b):
        pltpu.make_async_copy(
            row_bufs[b], out_hbm.at[pl.ds(base, CHUNK)], ssems[b]).wait()

    fire(0, 0)
    fire(1, 1)
    wait_gather(0); start_store(0, 0); fire(2, 2)
    wait_gather(1); start_store(1, 1); fire(3, 3)
    wait_gather(2); start_store(2, 2); wait_store(0); fire(4, 0)
    wait_gather(3); start_store(3, 3); wait_store(1); fire(5, 1)

    @pl.loop(NBUF, N_CHUNKS - 2, step=NBUF)
    def _(g):
        for b in range(NBUF):
            c = g + b
            wait_gather(b)
            start_store(c, b)
            bpf = (b + 2) % NBUF
            wait_store(bpf)
            fire(c + 2, bpf)

    wait_gather(0); start_store(N_CHUNKS - 2, 0)
    wait_gather(1); start_store(N_CHUNKS - 1, 1)
    wait_store(2); wait_store(3); wait_store(0); wait_store(1)


def kernel(word_ids, table):
    idx = word_ids.reshape(B).astype(jnp.int32)
    out = _embed(idx, table)
    return out.reshape(BATCH, HIST, D)


# speedup vs baseline: 8.0412x; 1.0290x over previous
import functools

import jax
import jax.numpy as jnp
from jax import lax
from jax.experimental import pallas as pl
from jax.experimental.pallas import tpu as pltpu
from jax.experimental.pallas import tpu_sc as plsc

BATCH = 1024
HIST = 200
D = 128
B = BATCH * HIST
NC = 2
NS = 16
NW = NC * NS
B_PER_W = B // NW
CHUNK = 128
N_CHUNKS = B_PER_W // CHUNK
NBUF = 5
AHEAD = 3

_mesh = plsc.VectorSubcoreMesh(core_axis_name="c", subcore_axis_name="s")


@functools.partial(
    pl.kernel,
    mesh=_mesh,
    out_type=jax.ShapeDtypeStruct((B, D), jnp.float32),
    scratch_types=(
        [pltpu.VMEM((B_PER_W,), jnp.int32)]
        + [pltpu.VMEM((CHUNK, D), jnp.float32) for _ in range(NBUF)]
        + [pltpu.SemaphoreType.DMA for _ in range(2 * NBUF)]
    ),
)
def _embed(idx_hbm, table_hbm, out_hbm, idx_all, *scr):
    row_bufs = scr[0:NBUF]
    gsems = scr[NBUF:2 * NBUF]
    ssems = scr[2 * NBUF:3 * NBUF]

    wid = lax.axis_index("s") * NC + lax.axis_index("c")
    base = wid * B_PER_W

    pltpu.sync_copy(idx_hbm.at[pl.ds(base, B_PER_W)], idx_all)

    def fire(c, b):
        idx_slice = idx_all.at[pl.ds(c * CHUNK, CHUNK)]
        pltpu.async_copy(table_hbm.at[idx_slice], row_bufs[b], gsems[b])

    def wait_gather(b):
        pltpu.make_async_copy(
            table_hbm.at[idx_all.at[pl.ds(0, CHUNK)]], row_bufs[b],
            gsems[b]).wait()

    def start_store(c, b):
        off = base + c * CHUNK
        pltpu.async_copy(row_bufs[b], out_hbm.at[pl.ds(off, CHUNK)], ssems[b])

    def wait_store(b):
        pltpu.make_async_copy(
            row_bufs[b], out_hbm.at[pl.ds(base, CHUNK)], ssems[b]).wait()

    for c in range(AHEAD):
        fire(c, c)

    @pl.loop(0, N_CHUNKS, step=NBUF)
    def _(g):
        for b in range(NBUF):
            c = g + b
            wait_gather(b)
            start_store(c, b)
            pf = c + AHEAD
            bpf = (b + AHEAD) % NBUF

            @pl.when(pf >= NBUF)
            def _():
                wait_store(bpf)

            @pl.when(pf < N_CHUNKS)
            def _():
                fire(pf, bpf)

    for c in range(N_CHUNKS - AHEAD + 1, N_CHUNKS):
        wait_store(c % NBUF)


def kernel(word_ids, table):
    idx = word_ids.reshape(B).astype(jnp.int32)
    out = _embed(idx, table)
    return out.reshape(BATCH, HIST, D)


# speedup vs baseline: 8.1133x; 1.0090x over previous
import functools

import jax
import jax.numpy as jnp
from jax import lax
from jax.experimental import pallas as pl
from jax.experimental.pallas import tpu as pltpu
from jax.experimental.pallas import tpu_sc as plsc

BATCH = 1024
HIST = 200
D = 128
B = BATCH * HIST
NC = 2
NS = 16
NW = NC * NS
B_PER_W = B // NW
CHUNK = 64
N_CHUNKS = B_PER_W // CHUNK
NBUF = 10
AHEAD = 6

_mesh = plsc.VectorSubcoreMesh(core_axis_name="c", subcore_axis_name="s")


@functools.partial(
    pl.kernel,
    mesh=_mesh,
    out_type=jax.ShapeDtypeStruct((B, D), jnp.float32),
    scratch_types=(
        [pltpu.VMEM((B_PER_W,), jnp.int32)]
        + [pltpu.VMEM((CHUNK, D), jnp.float32) for _ in range(NBUF)]
        + [pltpu.SemaphoreType.DMA for _ in range(2 * NBUF)]
    ),
)
def _embed(idx_hbm, table_hbm, out_hbm, idx_all, *scr):
    row_bufs = scr[0:NBUF]
    gsems = scr[NBUF:2 * NBUF]
    ssems = scr[2 * NBUF:3 * NBUF]

    wid = lax.axis_index("s") * NC + lax.axis_index("c")
    base = wid * B_PER_W

    pltpu.sync_copy(idx_hbm.at[pl.ds(base, B_PER_W)], idx_all)

    def fire(c, b):
        idx_slice = idx_all.at[pl.ds(c * CHUNK, CHUNK)]
        pltpu.async_copy(table_hbm.at[idx_slice], row_bufs[b], gsems[b])

    def wait_gather(b):
        pltpu.make_async_copy(
            table_hbm.at[idx_all.at[pl.ds(0, CHUNK)]], row_bufs[b],
            gsems[b]).wait()

    def start_store(c, b):
        off = base + c * CHUNK
        pltpu.async_copy(row_bufs[b], out_hbm.at[pl.ds(off, CHUNK)], ssems[b])

    def wait_store(b):
        pltpu.make_async_copy(
            row_bufs[b], out_hbm.at[pl.ds(base, CHUNK)], ssems[b]).wait()

    for c in range(AHEAD):
        fire(c, c)

    @pl.loop(0, N_CHUNKS, step=NBUF)
    def _(g):
        for b in range(NBUF):
            c = g + b
            wait_gather(b)
            start_store(c, b)
            pf = c + AHEAD
            bpf = (b + AHEAD) % NBUF

            @pl.when(pf >= NBUF)
            def _():
                wait_store(bpf)

            @pl.when(pf < N_CHUNKS)
            def _():
                fire(pf, bpf)

    for c in range(N_CHUNKS - (NBUF - AHEAD), N_CHUNKS):
        wait_store(c % NBUF)


def kernel(word_ids, table):
    idx = word_ids.reshape(B).astype(jnp.int32)
    out = _embed(idx, table)
    return out.reshape(BATCH, HIST, D)
